# baseline (speedup 1.0000x reference)
import functools

import jax
import jax.numpy as jnp
from jax import lax
from jax.experimental import pallas as pl
from jax.experimental.pallas import tpu as pltpu
from jax.experimental.pallas import tpu_sc as plsc

DIM = 64
RVQ = 4
NC = 2
NS = 16
NW = NC * NS
IDX_PER_CHUNK = 400
OUT_PER_CHUNK = IDX_PER_CHUNK // RVQ
GSPLIT = ((0, 128), (128, 128), (256, 128), (384, 16))


def _build(B, L):
    n_chunks = B * 2
    assert n_chunks % NW == 0
    niters = n_chunks // NW
    assert niters % 2 == 0
    mesh = plsc.VectorSubcoreMesh(core_axis_name="c", subcore_axis_name="s")

    @functools.partial(
        pl.kernel,
        out_type=jax.ShapeDtypeStruct((B, L, DIM), jnp.float32),
        mesh=mesh,
        scratch_types=[
            pltpu.VMEM((2, IDX_PER_CHUNK), jnp.int32),
            pltpu.VMEM((2, IDX_PER_CHUNK, DIM), jnp.float32),
            pltpu.VMEM((2, OUT_PER_CHUNK, DIM), jnp.float32),
            pltpu.SemaphoreType.DMA,
            pltpu.SemaphoreType.DMA,
            pltpu.SemaphoreType.DMA,
            pltpu.SemaphoreType.DMA,
            pltpu.SemaphoreType.DMA,
            pltpu.SemaphoreType.DMA,
        ],
        compiler_params=pltpu.CompilerParams(use_tc_tiling_on_sc=False),
    )
    def k(x_hbm, emb_hbm, out_hbm, idx_v, g_v, out_v,
          si0, si1, sg0, sg1, so0, so1):
        sem_i, sem_g, sem_o = (si0, si1), (sg0, sg1), (so0, so1)
        wid = lax.axis_index("s") * NC + lax.axis_index("c")
        base = wid * niters

        def fire_gathers(ch, b):
            for off, ln in GSPLIT:
                pltpu.make_async_copy(
                    emb_hbm.at[idx_v.at[b, pl.ds(off, ln)]],
                    g_v.at[b, pl.ds(off, ln)],
                    sem_g[b],
                ).start()

        def wait_gathers(b):
            for off, ln in GSPLIT:
                pltpu.make_async_copy(
                    emb_hbm.at[idx_v.at[b, pl.ds(off, ln)]],
                    g_v.at[b, pl.ds(off, ln)],
                    sem_g[b],
                ).wait()

        def idx_copy(ch, b):
            br = ch // 2
            h = ch % 2
            return pltpu.make_async_copy(
                x_hbm.at[br, pl.ds(h * IDX_PER_CHUNK, IDX_PER_CHUNK)],
                idx_v.at[b],
                sem_i[b],
            )

        def out_copy(ch, b):
            br = ch // 2
            h = ch % 2
            return pltpu.make_async_copy(
                out_v.at[b],
                out_hbm.at[br, pl.ds(h * OUT_PER_CHUNK, OUT_PER_CHUNK)],
                sem_o[b],
            )

        pltpu.sync_copy(
            x_hbm.at[base // 2, pl.ds((base % 2) * IDX_PER_CHUNK, IDX_PER_CHUNK)],
            idx_v.at[0],
        )
        fire_gathers(base, 0)
        idx_copy(base + 1, 1).start()

        def compute(b):
            def block(t, off_k, row0):
                vals = [
                    [
                        g_v[b, off_k + 4 * t + q, pl.ds(d * 16, 16)]
                        for q in range(RVQ)
                    ]
                    for d in range(DIM // 16)
                ]
                for d in range(DIM // 16):
                    v0, v1, v2, v3 = vals[d]
                    out_v[b, row0 + t, pl.ds(d * 16, 16)] = (v0 + v1) + (v2 + v3)

            for kk in range(3):
                def body(t, c2, kk=kk):
                    block(t, kk * 128, kk * 32)
                    return c2
                lax.fori_loop(0, 32, body, 0)
            for t in range(4):
                block(t, 384, 96)

        def outer(j, carry):
            for b in range(2):
                i = 2 * j + b
                ch = base + i
                nb = 1 - b

                @pl.when(i + 1 < niters)
                def _():
                    idx_copy(ch + 1, nb).wait()
                    fire_gathers(ch + 1, nb)

                wait_gathers(b)

                @pl.when(i + 2 < niters)
                def _():
                    idx_copy(ch + 2, b).start()

                @pl.when(i >= 2)
                def _():
                    out_copy(ch - 2, b).wait()

                compute(b)
                out_copy(ch, b).start()
            return carry

        lax.fori_loop(0, niters // 2, outer, 0)
        out_copy(base + niters - 2, 0).wait()
        out_copy(base + niters - 1, 1).wait()

    return k


def kernel(x, emb):
    B, W = x.shape
    L = W // RVQ
    assert W == 2 * IDX_PER_CHUNK
    return _build(B, L)(x, emb)
